# baseline (speedup 1.0000x reference)
import functools

import jax
import jax.numpy as jnp
from jax import lax
from jax.experimental import pallas as pl
from jax.experimental.pallas import tpu as pltpu
from jax.experimental.pallas import tpu_sc as plsc

_L = 16


def _make_embed_kernel(B, S, D, n_workers, chunk):
    pos_per_w = S // n_workers
    halves = pos_per_w // chunk
    n_chunks = B * halves
    vregs_per_chunk = chunk * (D // _L)

    mesh = plsc.VectorSubcoreMesh(core_axis_name="c", subcore_axis_name="s")

    @functools.partial(
        pl.kernel,
        mesh=mesh,
        out_type=jax.ShapeDtypeStruct((B * S, D), jnp.float32),
        scratch_types=[
            pltpu.VMEM((B * pos_per_w,), jnp.int32),
            pltpu.VMEM((pos_per_w, D), jnp.float32),
            pltpu.VMEM((chunk, D), jnp.float32),
            pltpu.VMEM((chunk, D), jnp.float32),
            pltpu.SemaphoreType.DMA,
        ],
    )
    def embed(idx_hbm, tok_hbm, pos_hbm, out_hbm, idx_v, pos_v, tok0, tok1, sem):
        nc = 2
        wid = lax.axis_index("s") * nc + lax.axis_index("c")
        pos_base = wid * pos_per_w

        for b in range(B):
            pltpu.sync_copy(
                idx_hbm.at[pl.ds(b * S + pos_base, pos_per_w)],
                idx_v.at[pl.ds(b * pos_per_w, pos_per_w)],
            )
        pltpu.sync_copy(pos_hbm.at[pl.ds(pos_base, pos_per_w)], pos_v)

        bufs = (tok0, tok1)

        def idx_slice(ci):
            return idx_v.at[pl.ds(ci * chunk, chunk)]

        def out_slice(ci):
            b, h = divmod(ci, halves)
            return out_hbm.at[pl.ds(b * S + pos_base + h * chunk, chunk)]

        pltpu.async_copy(tok_hbm.at[idx_slice(0)], bufs[0], sem)
        for ci in range(n_chunks):
            buf = bufs[ci % 2]
            pltpu.make_async_copy(tok_hbm.at[idx_slice(ci)], buf, sem).wait()
            if ci + 1 < n_chunks:
                pltpu.async_copy(
                    tok_hbm.at[idx_slice(ci + 1)], bufs[(ci + 1) % 2], sem
                )
            prow = (ci % halves) * chunk

            def add_row(r, _, buf=buf, prow=prow):
                for c in range(D // _L):
                    sl = pl.ds(c * _L, _L)
                    buf[r, sl] = buf[r, sl] + pos_v[prow + r, sl]
                return 0

            lax.fori_loop(0, chunk, add_row, 0)
            pltpu.sync_copy(buf, out_slice(ci))

    return embed


def kernel(inputs, token_table, pos_table):
    B, S = inputs.shape
    V, D = token_table.shape
    flat_idx = inputs.reshape(B * S).astype(jnp.int32)
    embed = _make_embed_kernel(B, S, D, n_workers=32, chunk=32)
    out = embed(flat_idx, token_table, pos_table)
    return out.reshape(B, S, D)


# speedup vs baseline: 1.1145x; 1.1145x over previous
import functools

import jax
import jax.numpy as jnp
from jax import lax
from jax.experimental import pallas as pl
from jax.experimental.pallas import tpu as pltpu
from jax.experimental.pallas import tpu_sc as plsc

_L = 16


def _make_embed_kernel(B, S, D, n_workers, chunk):
    pos_per_w = S // n_workers
    halves = pos_per_w // chunk
    n_chunks = B * halves
    vregs_per_chunk = chunk * (D // _L)

    mesh = plsc.VectorSubcoreMesh(core_axis_name="c", subcore_axis_name="s")

    @functools.partial(
        pl.kernel,
        mesh=mesh,
        out_type=jax.ShapeDtypeStruct((B * S, D), jnp.float32),
        scratch_types=[
            pltpu.VMEM((B * pos_per_w,), jnp.int32),
            pltpu.VMEM((pos_per_w, D), jnp.float32),
            pltpu.VMEM((chunk, D), jnp.float32),
            pltpu.VMEM((chunk, D), jnp.float32),
            pltpu.SemaphoreType.DMA,
            pltpu.SemaphoreType.DMA,
        ],
    )
    def embed(
        idx_hbm, tok_hbm, pos_hbm, out_hbm, idx_v, pos_v, tok0, tok1, gsem, wsem
    ):
        nc = 2
        wid = lax.axis_index("s") * nc + lax.axis_index("c")
        pos_base = wid * pos_per_w

        for b in range(B):
            pltpu.sync_copy(
                idx_hbm.at[pl.ds(b * S + pos_base, pos_per_w)],
                idx_v.at[pl.ds(b * pos_per_w, pos_per_w)],
            )
        pltpu.sync_copy(pos_hbm.at[pl.ds(pos_base, pos_per_w)], pos_v)

        bufs = (tok0, tok1)

        def idx_slice(ci):
            return idx_v.at[pl.ds(ci * chunk, chunk)]

        def out_slice(ci):
            b, h = divmod(ci, halves)
            return out_hbm.at[pl.ds(b * S + pos_base + h * chunk, chunk)]

        pltpu.async_copy(tok_hbm.at[idx_slice(0)], bufs[0], gsem)
        for ci in range(n_chunks):
            buf = bufs[ci % 2]
            pltpu.make_async_copy(tok_hbm.at[idx_slice(ci)], buf, gsem).wait()
            if ci + 1 < n_chunks:
                if ci >= 1:
                    pltpu.make_async_copy(
                        bufs[(ci + 1) % 2], out_slice(ci - 1), wsem
                    ).wait()
                pltpu.async_copy(
                    tok_hbm.at[idx_slice(ci + 1)], bufs[(ci + 1) % 2], gsem
                )
            prow = (ci % halves) * chunk

            def add_row(r, _, buf=buf, prow=prow):
                for c in range(D // _L):
                    sl = pl.ds(c * _L, _L)
                    plsc.addupdate(buf.at[r, sl], pos_v[prow + r, sl])
                return 0

            lax.fori_loop(0, chunk, add_row, 0)
            if ci + 1 < n_chunks:
                pltpu.async_copy(buf, out_slice(ci), wsem)
            else:
                pltpu.make_async_copy(
                    bufs[(ci - 1) % 2], out_slice(ci - 1), wsem
                ).wait()
                pltpu.sync_copy(buf, out_slice(ci))

    return embed


def kernel(inputs, token_table, pos_table):
    B, S = inputs.shape
    V, D = token_table.shape
    flat_idx = inputs.reshape(B * S).astype(jnp.int32)
    embed = _make_embed_kernel(B, S, D, n_workers=32, chunk=32)
    out = embed(flat_idx, token_table, pos_table)
    return out.reshape(B, S, D)


# speedup vs baseline: 1.1839x; 1.0623x over previous
import functools

import jax
import jax.numpy as jnp
from jax import lax
from jax.experimental import pallas as pl
from jax.experimental.pallas import tpu as pltpu
from jax.experimental.pallas import tpu_sc as plsc

_L = 16


def _make_embed_kernel(B, S, D, n_workers, chunk):
    pos_per_w = S // n_workers
    halves = pos_per_w // chunk
    n_chunks = B * halves
    vregs_per_chunk = chunk * (D // _L)

    mesh = plsc.VectorSubcoreMesh(core_axis_name="c", subcore_axis_name="s")

    @functools.partial(
        pl.kernel,
        mesh=mesh,
        out_type=jax.ShapeDtypeStruct((B * S, D), jnp.float32),
        scratch_types=[
            pltpu.VMEM((B * pos_per_w,), jnp.int32),
            pltpu.VMEM((pos_per_w, D), jnp.float32),
            pltpu.VMEM((chunk, D), jnp.float32),
            pltpu.VMEM((chunk, D), jnp.float32),
            pltpu.VMEM((chunk, D), jnp.float32),
            pltpu.SemaphoreType.DMA,
            pltpu.SemaphoreType.DMA,
        ],
    )
    def embed(
        idx_hbm, tok_hbm, pos_hbm, out_hbm,
        idx_v, pos_v, tok0, tok1, tok2, gsem, wsem,
    ):
        nc = 2
        wid = lax.axis_index("s") * nc + lax.axis_index("c")
        pos_base = wid * pos_per_w

        for b in range(B):
            pltpu.sync_copy(
                idx_hbm.at[pl.ds(b * S + pos_base, pos_per_w)],
                idx_v.at[pl.ds(b * pos_per_w, pos_per_w)],
            )
        pltpu.sync_copy(pos_hbm.at[pl.ds(pos_base, pos_per_w)], pos_v)

        bufs = (tok0, tok1, tok2)

        def idx_slice(ci):
            return idx_v.at[pl.ds(ci * chunk, chunk)]

        def out_slice(ci):
            b, h = divmod(ci, halves)
            return out_hbm.at[pl.ds(b * S + pos_base + h * chunk, chunk)]

        nbuf = 3
        pltpu.async_copy(tok_hbm.at[idx_slice(0)], bufs[0], gsem)
        pltpu.async_copy(tok_hbm.at[idx_slice(1)], bufs[1], gsem)
        for ci in range(n_chunks):
            buf = bufs[ci % nbuf]
            pltpu.make_async_copy(tok_hbm.at[idx_slice(ci)], buf, gsem).wait()
            if ci + 2 < n_chunks:
                if ci >= 1:
                    pltpu.make_async_copy(
                        bufs[(ci + 2) % nbuf], out_slice(ci - 1), wsem
                    ).wait()
                pltpu.async_copy(
                    tok_hbm.at[idx_slice(ci + 2)], bufs[(ci + 2) % nbuf], gsem
                )
            prow = (ci % halves) * chunk

            def add_row(r, _, buf=buf, prow=prow):
                for c in range(D // _L):
                    sl = pl.ds(c * _L, _L)
                    plsc.addupdate(buf.at[r, sl], pos_v[prow + r, sl])
                return 0

            lax.fori_loop(0, chunk, add_row, 0)
            pltpu.async_copy(buf, out_slice(ci), wsem)
        for ci in range(max(0, n_chunks - 3), n_chunks):
            pltpu.make_async_copy(
                bufs[ci % nbuf], out_slice(ci), wsem
            ).wait()

    return embed


def kernel(inputs, token_table, pos_table):
    B, S = inputs.shape
    V, D = token_table.shape
    flat_idx = inputs.reshape(B * S).astype(jnp.int32)
    embed = _make_embed_kernel(B, S, D, n_workers=32, chunk=32)
    out = embed(flat_idx, token_table, pos_table)
    return out.reshape(B, S, D)
